# baseline (speedup 1.0000x reference)
import functools

import jax
import jax.numpy as jnp
from jax import lax
from jax.experimental import pallas as pl
from jax.experimental.pallas import tpu as pltpu
from jax.experimental.pallas import tpu_sc as plsc

NUM_TEAMS = 100000
EMBED_DIM = 128
BATCH = 16384
FEAT2 = 2 * EMBED_DIM
N_WORKERS = 32
ROWS_PER_W = BATCH // N_WORKERS
CHUNK = 128
N_CHUNKS = ROWS_PER_W // CHUNK
N_UNITS = 2 * N_CHUNKS
NBUF = 6
EPS = 1e-5

N_TC_CHUNKS = 4
CROWS = BATCH // N_TC_CHUNKS
CROWS128 = CROWS // 128


def _sc_gather_body(table, idx1, idx2, feats, idx_v, bufs, *sems):
    gsems = sems[:NBUF]
    wsems = sems[NBUF:]
    wid = lax.axis_index("s") * 2 + lax.axis_index("c")
    base = wid * ROWS_PER_W
    irow = wid * N_CHUNKS
    pltpu.sync_copy(idx1.at[pl.ds(irow, N_CHUNKS)], idx_v.at[pl.ds(0, N_CHUNKS)])
    pltpu.sync_copy(
        idx2.at[pl.ds(irow, N_CHUNKS)], idx_v.at[pl.ds(N_CHUNKS, N_CHUNKS)]
    )

    def buf_at(u):
        return bufs.at[pl.ds((u % NBUF) * CHUNK, CHUNK)]

    def feats_at(u):
        half, j = divmod(u, N_CHUNKS)
        return feats.at[
            pl.ds(base + j * CHUNK, CHUNK), pl.ds(half * EMBED_DIM, EMBED_DIM)
        ]

    gathers = {}
    writes = {}
    for u in range(min(NBUF, N_UNITS)):
        gathers[u] = pltpu.async_copy(table.at[idx_v.at[u]], buf_at(u), gsems[u % NBUF])
    for u in range(N_UNITS):
        gathers[u].wait()
        writes[u] = pltpu.async_copy(buf_at(u), feats_at(u), wsems[u % NBUF])
        if u + NBUF < N_UNITS:
            writes[u].wait()
            gathers[u + NBUF] = pltpu.async_copy(
                table.at[idx_v.at[u + NBUF]], buf_at(u + NBUF), gsems[(u + NBUF) % NBUF]
            )
    for u in range(max(0, N_UNITS - NBUF), N_UNITS):
        writes[u].wait()


_sc_gather = functools.partial(
    pl.kernel,
    mesh=plsc.VectorSubcoreMesh(core_axis_name="c", subcore_axis_name="s"),
    out_type=jax.ShapeDtypeStruct((BATCH, FEAT2), jnp.float32),
    scratch_types=[
        pltpu.VMEM((N_UNITS, CHUNK), jnp.int32),
        pltpu.VMEM((NBUF * CHUNK, EMBED_DIM), jnp.float32),
    ]
    + [pltpu.SemaphoreType.DMA] * (2 * NBUF),
)(_sc_gather_body)


def _tc_bn_body(feats_hbm, s2d_ref, gamma_ref, beta_ref, w_ref, b_ref, out_ref, x_ref, z_ref, sems):
    copies = []
    for i in range(N_TC_CHUNKS):
        cp = pltpu.make_async_copy(
            feats_hbm.at[pl.ds(i * CROWS128, CROWS128)],
            x_ref.at[pl.ds(i * CROWS128, CROWS128)],
            sems.at[i],
        )
        cp.start()
        copies.append(cp)
    wv = w_ref[0, :]
    gw = gamma_ref[...] * wv
    bconst = b_ref[0] + jnp.sum(beta_ref[...] * wv)
    ssum = jnp.zeros((FEAT2,), jnp.float32)
    ssq = jnp.zeros((FEAT2,), jnp.float32)
    for i in range(N_TC_CHUNKS):
        copies[i].wait()
        x = x_ref[pl.ds(i * CROWS128, CROWS128)]
        ssum = ssum + jnp.sum(jnp.sum(x, axis=0), axis=0)
        ssq = ssq + jnp.sum(jnp.sum(x * x, axis=0), axis=0)
    s = s2d_ref[...]
    inv_n = 1.0 / BATCH
    smean = jnp.sum(s) * inv_n
    svar = jnp.sum(s * s) * inv_n - smean * smean
    mean = ssum * inv_n
    var = ssq * inv_n - mean * mean
    c = gw[:FEAT2] * lax.rsqrt(var + EPS)
    cs = gw[FEAT2] * lax.rsqrt(svar + EPS)
    const = bconst - jnp.sum(c * mean) - cs * smean
    for i in range(N_TC_CHUNKS):
        x = x_ref[pl.ds(i * CROWS128, CROWS128)]
        z_ref[pl.ds(i * CROWS128, CROWS128), :] = jnp.sum(x * c, axis=2)
    zz = z_ref[...] + s * cs + const
    out_ref[...] = jax.nn.sigmoid(zz)


def _tc_bn(feats3, s2d, gamma, beta, W, b):
    return pl.pallas_call(
        _tc_bn_body,
        in_specs=[
            pl.BlockSpec(memory_space=pltpu.MemorySpace.HBM),
            pl.BlockSpec(memory_space=pltpu.VMEM),
            pl.BlockSpec(memory_space=pltpu.VMEM),
            pl.BlockSpec(memory_space=pltpu.VMEM),
            pl.BlockSpec(memory_space=pltpu.VMEM),
            pl.BlockSpec(memory_space=pltpu.VMEM),
        ],
        out_specs=pl.BlockSpec(memory_space=pltpu.VMEM),
        out_shape=jax.ShapeDtypeStruct((128, 128), jnp.float32),
        scratch_shapes=[
            pltpu.VMEM((128, 128, FEAT2), jnp.float32),
            pltpu.VMEM((128, 128), jnp.float32),
            pltpu.SemaphoreType.DMA((N_TC_CHUNKS,)),
        ],
    )(feats3, s2d, gamma, beta, W, b)


def kernel(idsTensor, table, gamma, beta, W, b):
    idx1 = idsTensor[:, 0].astype(jnp.int32).reshape(128, 128)
    idx2 = idsTensor[:, 1].astype(jnp.int32).reshape(128, 128)
    s2d = idsTensor[:, 2].reshape(128, 128)
    feats = _sc_gather(table, idx1, idx2)
    feats3 = feats.reshape(128, 128, FEAT2)
    out = _tc_bn(feats3, s2d, gamma, beta, W, b)
    return out.reshape(BATCH, 1)
